# baseline (device time: 58440 ns/iter reference)
import jax
import jax.numpy as jnp
from jax import lax
from jax.experimental import pallas as pl
from jax.experimental.pallas import tpu as pltpu

N_DEV = 4
SQ = 1024
SKV = 1024
HQ = 8
DH = 128
D = HQ * DH
G = SQ // 4
SCALE = 0.08838834764831843


def kernel(x, Wq, K_ext, V_ext, Wo):
    def body(x_ref, wq_ref, k_hbm, v_hbm, wo_hbm, out_hbm, *scr):
        kv_ref, vv_ref, wo_ref, l_ref, out_vm = scr[0:5]
        pack1 = scr[5:9]
        pack2 = scr[9:13]
        lp1 = scr[13:17]
        lp2 = scr[17:21]
        rctx1 = scr[21:25]
        rctx2 = scr[25:29]
        rl1 = scr[29:33]
        rl2 = scr[33:37]
        load_sems, send_sems, recv_sems, out_sems = scr[37:41]

        p = lax.axis_index("i")
        p_y = p ^ 1
        p_x = 3 - p

        ldk = pltpu.make_async_copy(k_hbm, kv_ref, load_sems.at[0])
        ldv = pltpu.make_async_copy(v_hbm, vv_ref, load_sems.at[1])
        ldw = pltpu.make_async_copy(wo_hbm, wo_ref, load_sems.at[2])
        ldk.start()
        ldv.start()
        ldw.start()

        bar = pltpu.get_barrier_semaphore()
        for nbr in (p_x, p_y):
            pl.semaphore_signal(bar, inc=1, device_id=(nbr,),
                                device_id_type=pl.DeviceIdType.MESH)
        pl.semaphore_wait(bar, 2)

        def exchange(src, dst, partner, sem_idx):
            return pltpu.make_async_remote_copy(
                src_ref=src, dst_ref=dst,
                send_sem=send_sems.at[sem_idx],
                recv_sem=recv_sems.at[sem_idx],
                device_id=(partner,),
                device_id_type=pl.DeviceIdType.MESH,
            )

        def group_rows(m):
            return m.reshape(4, 4, 64, D).transpose(1, 0, 2, 3).reshape(SQ, D)

        xg = group_rows(x_ref[0])
        wqb = wq_ref[...]

        ldk.wait()
        ldv.wait()
        kb = group_rows(kv_ref[0].reshape(SKV, D))
        vb = group_rows(vv_ref[0].reshape(SKV, D))

        link1 = [p_x, p_y, p_x, p_y]
        link2 = [p_y, p_x, p_y, p_x]
        r1 = [None] * 4
        r2 = [None] * 4
        outcp = []

        def rows(q):
            return pl.ds(q * G, G)

        for q in range(4):
            qq = lax.dot_general(xg[q * G:(q + 1) * G, :], wqb,
                                 (((1,), (0,)), ((), ())),
                                 preferred_element_type=jnp.float32)
            qqb = (qq * SCALE).astype(jnp.bfloat16)
            lrows = []
            for h in range(HQ):
                cols = pl.ds(h * DH, DH)
                kh = kb[q * G:(q + 1) * G, h * DH:(h + 1) * DH]
                vh = vb[q * G:(q + 1) * G, h * DH:(h + 1) * DH]
                s = lax.dot_general(qqb[:, h * DH:(h + 1) * DH], kh,
                                    (((1,), (1,)), ((), ())),
                                    preferred_element_type=jnp.float32)
                w = jnp.exp(s)
                lrows.append(w.sum(axis=1))
                ctx = lax.dot_general(
                    w.astype(jnp.bfloat16), vh, (((1,), (0,)), ((), ())),
                    preferred_element_type=jnp.float32)
                pack1[q][:, cols] = ctx.astype(jnp.bfloat16)
            l_q = jnp.stack(lrows)
            l_ref[:, rows(q)] = l_q
            lp1[q][...] = l_q
            r1[q] = [exchange(pack1[q], rctx1[q], link1[q], q),
                     exchange(lp1[q], rl1[q], link1[q], 8 + q)]
            for rdma in r1[q]:
                rdma.start()

        for q in range(4):
            for rdma in r1[q]:
                rdma.wait()
            acc_q = (pack1[q][...].astype(jnp.float32)
                     + rctx1[q][...].astype(jnp.float32))
            l_q = l_ref[:, rows(q)] + rl1[q][...]
            l_ref[:, rows(q)] = l_q
            pack2[q][...] = acc_q.astype(jnp.bfloat16)
            lp2[q][...] = l_q
            r2[q] = [exchange(pack2[q], rctx2[q], link2[q], 4 + q),
                     exchange(lp2[q], rl2[q], link2[q], 12 + q)]
            for rdma in r2[q]:
                rdma.start()

        ldw.wait()
        wob = wo_ref[...]

        for q in range(4):
            for rdma in r2[q]:
                rdma.wait()
            acc_q = (pack2[q][...].astype(jnp.float32)
                     + rctx2[q][...].astype(jnp.float32))
            l_q = l_ref[:, rows(q)] + rl2[q][...]
            ctx_q = (acc_q.reshape(G, HQ, DH)
                     / jnp.transpose(l_q)[:, :, None]).reshape(G, D)
            out_q = lax.dot_general(
                ctx_q.astype(jnp.bfloat16), wob, (((1,), (0,)), ((), ())),
                preferred_element_type=jnp.float32)
            out_vm[rows(q), :] = out_q
            for a in range(4):
                cp = pltpu.make_async_copy(
                    out_vm.at[pl.ds(q * G + 64 * a, 64)],
                    out_hbm.at[0, pl.ds(256 * a + 64 * q, 64)],
                    out_sems.at[4 * q + a])
                cp.start()
                outcp.append(cp)

        for cp in outcp:
            cp.wait()

    xb = x.astype(jnp.bfloat16)
    wqb = Wq.astype(jnp.bfloat16)
    kb = K_ext.astype(jnp.bfloat16)
    vb = V_ext.astype(jnp.bfloat16)
    wob = Wo.astype(jnp.bfloat16)

    return pl.pallas_call(
        body,
        out_shape=jax.ShapeDtypeStruct((1, SQ, D), jnp.float32),
        in_specs=[
            pl.BlockSpec(memory_space=pltpu.MemorySpace.VMEM),
            pl.BlockSpec(memory_space=pltpu.MemorySpace.VMEM),
            pl.BlockSpec(memory_space=pltpu.MemorySpace.HBM),
            pl.BlockSpec(memory_space=pltpu.MemorySpace.HBM),
            pl.BlockSpec(memory_space=pltpu.MemorySpace.HBM),
        ],
        out_specs=pl.BlockSpec(memory_space=pltpu.MemorySpace.HBM),
        scratch_shapes=(
            [
                pltpu.VMEM((1, SKV, HQ, DH), jnp.bfloat16),
                pltpu.VMEM((1, SKV, HQ, DH), jnp.bfloat16),
                pltpu.VMEM((D, D), jnp.bfloat16),
                pltpu.VMEM((HQ, SQ), jnp.float32),
                pltpu.VMEM((SQ, D), jnp.float32),
            ]
            + [pltpu.VMEM((G, D), jnp.bfloat16) for _ in range(8)]
            + [pltpu.VMEM((HQ, G), jnp.float32) for _ in range(8)]
            + [pltpu.VMEM((G, D), jnp.bfloat16) for _ in range(8)]
            + [pltpu.VMEM((HQ, G), jnp.float32) for _ in range(8)]
            + [
                pltpu.SemaphoreType.DMA((3,)),
                pltpu.SemaphoreType.DMA((16,)),
                pltpu.SemaphoreType.DMA((16,)),
                pltpu.SemaphoreType.DMA((16,)),
            ]
        ),
        compiler_params=pltpu.CompilerParams(
            collective_id=0, vmem_limit_bytes=64 * 1024 * 1024),
    )(xb, wqb, kb, vb, wob)


# device time: 46487 ns/iter; 1.2571x vs baseline; 1.2571x over previous
import jax
import jax.numpy as jnp
from jax import lax
from jax.experimental import pallas as pl
from jax.experimental.pallas import tpu as pltpu

N_DEV = 4
SQ = 1024
SKV = 1024
HQ = 8
DH = 128
D = HQ * DH
G = SQ // 4
SCALE = 0.08838834764831843


def kernel(x, Wq, K_ext, V_ext, Wo):
    def body(x_ref, wq_ref, k_hbm, v_hbm, wo_hbm, out_hbm, *scr):
        kv_ref, vv_ref, wo_ref, l_ref, out_vm = scr[0:5]
        pack1 = scr[5:9]
        pack2 = scr[9:13]
        lp1 = scr[13:17]
        lp2 = scr[17:21]
        rctx1 = scr[21:25]
        rctx2 = scr[25:29]
        rl1 = scr[29:33]
        rl2 = scr[33:37]
        load_sems, send_sems, recv_sems, out_sems = scr[37:41]

        p = lax.axis_index("i")
        p_y = p ^ 1
        p_x = 3 - p

        ldk = pltpu.make_async_copy(k_hbm, kv_ref, load_sems.at[0])
        ldv = pltpu.make_async_copy(v_hbm, vv_ref, load_sems.at[1])
        ldw = pltpu.make_async_copy(wo_hbm, wo_ref, load_sems.at[2])
        ldk.start()
        ldv.start()
        ldw.start()

        bar = pltpu.get_barrier_semaphore()
        for nbr in (p_x, p_y):
            pl.semaphore_signal(bar, inc=1, device_id=(nbr,),
                                device_id_type=pl.DeviceIdType.MESH)
        pl.semaphore_wait(bar, 2)

        def exchange(src, dst, partner, sem_idx):
            return pltpu.make_async_remote_copy(
                src_ref=src, dst_ref=dst,
                send_sem=send_sems.at[sem_idx],
                recv_sem=recv_sems.at[sem_idx],
                device_id=(partner,),
                device_id_type=pl.DeviceIdType.MESH,
            )

        def group_rows(m):
            return m.reshape(4, 4, 64, D).transpose(1, 0, 2, 3).reshape(SQ, D)

        xg = group_rows(x_ref[0].astype(jnp.bfloat16))
        wqb = wq_ref[...].astype(jnp.bfloat16)

        ldk.wait()
        ldv.wait()
        kb = group_rows(kv_ref[0].reshape(SKV, D).astype(jnp.bfloat16))
        vb = group_rows(vv_ref[0].reshape(SKV, D).astype(jnp.bfloat16))

        link1 = [p_x, p_y, p_x, p_y]
        link2 = [p_y, p_x, p_y, p_x]
        r1 = [None] * 4
        r2 = [None] * 4
        outcp = []

        def rows(q):
            return pl.ds(q * G, G)

        for q in range(4):
            qq = lax.dot_general(xg[q * G:(q + 1) * G, :], wqb,
                                 (((1,), (0,)), ((), ())),
                                 preferred_element_type=jnp.float32)
            qqb = (qq * SCALE).astype(jnp.bfloat16)
            lrows = []
            for h in range(HQ):
                cols = pl.ds(h * DH, DH)
                kh = kb[q * G:(q + 1) * G, h * DH:(h + 1) * DH]
                vh = vb[q * G:(q + 1) * G, h * DH:(h + 1) * DH]
                s = lax.dot_general(qqb[:, h * DH:(h + 1) * DH], kh,
                                    (((1,), (1,)), ((), ())),
                                    preferred_element_type=jnp.float32)
                w = jnp.exp(s)
                lrows.append(w.sum(axis=1))
                ctx = lax.dot_general(
                    w.astype(jnp.bfloat16), vh, (((1,), (0,)), ((), ())),
                    preferred_element_type=jnp.float32)
                pack1[q][:, cols] = ctx.astype(jnp.bfloat16)
            l_q = jnp.stack(lrows)
            l_ref[:, rows(q)] = l_q
            lp1[q][...] = l_q
            r1[q] = [exchange(pack1[q], rctx1[q], link1[q], q),
                     exchange(lp1[q], rl1[q], link1[q], 8 + q)]
            for rdma in r1[q]:
                rdma.start()

        for q in range(4):
            for rdma in r1[q]:
                rdma.wait()
            acc_q = (pack1[q][...].astype(jnp.float32)
                     + rctx1[q][...].astype(jnp.float32))
            l_q = l_ref[:, rows(q)] + rl1[q][...]
            l_ref[:, rows(q)] = l_q
            pack2[q][...] = acc_q.astype(jnp.bfloat16)
            lp2[q][...] = l_q
            r2[q] = [exchange(pack2[q], rctx2[q], link2[q], 4 + q),
                     exchange(lp2[q], rl2[q], link2[q], 12 + q)]
            for rdma in r2[q]:
                rdma.start()

        ldw.wait()
        wob = wo_ref[...].astype(jnp.bfloat16)

        for q in range(4):
            for rdma in r2[q]:
                rdma.wait()
            acc_q = (pack2[q][...].astype(jnp.float32)
                     + rctx2[q][...].astype(jnp.float32))
            l_q = l_ref[:, rows(q)] + rl2[q][...]
            ctx_q = (acc_q.reshape(G, HQ, DH)
                     / jnp.transpose(l_q)[:, :, None]).reshape(G, D)
            out_q = lax.dot_general(
                ctx_q.astype(jnp.bfloat16), wob, (((1,), (0,)), ((), ())),
                preferred_element_type=jnp.float32)
            out_vm[rows(q), :] = out_q
            for a in range(4):
                cp = pltpu.make_async_copy(
                    out_vm.at[pl.ds(q * G + 64 * a, 64)],
                    out_hbm.at[0, pl.ds(256 * a + 64 * q, 64)],
                    out_sems.at[4 * q + a])
                cp.start()
                outcp.append(cp)

        for cp in outcp:
            cp.wait()

    return pl.pallas_call(
        body,
        out_shape=jax.ShapeDtypeStruct((1, SQ, D), jnp.float32),
        in_specs=[
            pl.BlockSpec(memory_space=pltpu.MemorySpace.VMEM),
            pl.BlockSpec(memory_space=pltpu.MemorySpace.VMEM),
            pl.BlockSpec(memory_space=pltpu.MemorySpace.HBM),
            pl.BlockSpec(memory_space=pltpu.MemorySpace.HBM),
            pl.BlockSpec(memory_space=pltpu.MemorySpace.HBM),
        ],
        out_specs=pl.BlockSpec(memory_space=pltpu.MemorySpace.HBM),
        scratch_shapes=(
            [
                pltpu.VMEM((1, SKV, HQ, DH), jnp.float32),
                pltpu.VMEM((1, SKV, HQ, DH), jnp.float32),
                pltpu.VMEM((D, D), jnp.float32),
                pltpu.VMEM((HQ, SQ), jnp.float32),
                pltpu.VMEM((SQ, D), jnp.float32),
            ]
            + [pltpu.VMEM((G, D), jnp.bfloat16) for _ in range(8)]
            + [pltpu.VMEM((HQ, G), jnp.float32) for _ in range(8)]
            + [pltpu.VMEM((G, D), jnp.bfloat16) for _ in range(8)]
            + [pltpu.VMEM((HQ, G), jnp.float32) for _ in range(8)]
            + [
                pltpu.SemaphoreType.DMA((3,)),
                pltpu.SemaphoreType.DMA((16,)),
                pltpu.SemaphoreType.DMA((16,)),
                pltpu.SemaphoreType.DMA((16,)),
            ]
        ),
        compiler_params=pltpu.CompilerParams(
            collective_id=0, vmem_limit_bytes=64 * 1024 * 1024),
    )(x, Wq, K_ext, V_ext, Wo)


# device time: 44565 ns/iter; 1.3113x vs baseline; 1.0431x over previous
import jax
import jax.numpy as jnp
from jax import lax
from jax.experimental import pallas as pl
from jax.experimental.pallas import tpu as pltpu

N_DEV = 4
SQ = 1024
SKV = 1024
HQ = 8
DH = 128
D = HQ * DH
G = SQ // 4
SCALE = 0.08838834764831843


def kernel(x, Wq, K_ext, V_ext, Wo):
    def body(x_ref, wq_ref, k_hbm, v_hbm, wo_hbm, out_hbm, *scr):
        wo_ref, l_ref, out_vm = scr[0:3]
        kq = scr[3:7]
        vq = scr[7:11]
        pack1 = scr[11:15]
        pack2 = scr[15:19]
        lp1 = scr[19:23]
        lp2 = scr[23:27]
        rctx1 = scr[27:31]
        rctx2 = scr[31:35]
        rl1 = scr[35:39]
        rl2 = scr[39:43]
        load_sems, kv_sems, send_sems, recv_sems, out_sems = scr[43:48]

        p = lax.axis_index("i")
        p_y = p ^ 1
        p_x = 3 - p

        ldkv = [[], [], [], []]
        for q in range(4):
            for a in range(4):
                ck = pltpu.make_async_copy(
                    k_hbm.at[0, pl.ds(64 * (4 * a + q), 64)],
                    kq[q].at[pl.ds(64 * a, 64)],
                    kv_sems.at[8 * q + a])
                cv = pltpu.make_async_copy(
                    v_hbm.at[0, pl.ds(64 * (4 * a + q), 64)],
                    vq[q].at[pl.ds(64 * a, 64)],
                    kv_sems.at[8 * q + 4 + a])
                ck.start()
                cv.start()
                ldkv[q] += [ck, cv]
        ldw = pltpu.make_async_copy(wo_hbm, wo_ref, load_sems.at[2])
        ldw.start()

        bar = pltpu.get_barrier_semaphore()
        for nbr in (p_x, p_y):
            pl.semaphore_signal(bar, inc=1, device_id=(nbr,),
                                device_id_type=pl.DeviceIdType.MESH)
        pl.semaphore_wait(bar, 2)

        def exchange(src, dst, partner, sem_idx):
            return pltpu.make_async_remote_copy(
                src_ref=src, dst_ref=dst,
                send_sem=send_sems.at[sem_idx],
                recv_sem=recv_sems.at[sem_idx],
                device_id=(partner,),
                device_id_type=pl.DeviceIdType.MESH,
            )

        def group_rows(m):
            return m.reshape(4, 4, 64, D).transpose(1, 0, 2, 3).reshape(SQ, D)

        xg = group_rows(x_ref[0].astype(jnp.bfloat16))
        wqb = wq_ref[...].astype(jnp.bfloat16)

        link1 = [p_x, p_y, p_x, p_y]
        link2 = [p_y, p_x, p_y, p_x]
        r1 = [None] * 4
        r2 = [None] * 4
        outcp = []

        def rows(q):
            return pl.ds(q * G, G)

        for q in range(4):
            qq = lax.dot_general(xg[q * G:(q + 1) * G, :], wqb,
                                 (((1,), (0,)), ((), ())),
                                 preferred_element_type=jnp.float32)
            qqb = (qq * SCALE).astype(jnp.bfloat16)
            for cp in ldkv[q]:
                cp.wait()
            kbq = kq[q][...].reshape(G, D).astype(jnp.bfloat16)
            vbq = vq[q][...].reshape(G, D).astype(jnp.bfloat16)
            lrows = []
            for h in range(HQ):
                cols = pl.ds(h * DH, DH)
                kh = kbq[:, h * DH:(h + 1) * DH]
                vh = vbq[:, h * DH:(h + 1) * DH]
                s = lax.dot_general(qqb[:, h * DH:(h + 1) * DH], kh,
                                    (((1,), (1,)), ((), ())),
                                    preferred_element_type=jnp.float32)
                w = jnp.exp(s)
                lrows.append(w.sum(axis=1))
                ctx = lax.dot_general(
                    w.astype(jnp.bfloat16), vh, (((1,), (0,)), ((), ())),
                    preferred_element_type=jnp.float32)
                pack1[q][:, cols] = ctx.astype(jnp.bfloat16)
            l_q = jnp.stack(lrows)
            l_ref[:, rows(q)] = l_q
            lp1[q][...] = l_q
            r1[q] = [exchange(pack1[q], rctx1[q], link1[q], q),
                     exchange(lp1[q], rl1[q], link1[q], 8 + q)]
            for rdma in r1[q]:
                rdma.start()

        for q in range(4):
            for rdma in r1[q]:
                rdma.wait()
            acc_q = (pack1[q][...].astype(jnp.float32)
                     + rctx1[q][...].astype(jnp.float32))
            l_q = l_ref[:, rows(q)] + rl1[q][...]
            l_ref[:, rows(q)] = l_q
            pack2[q][...] = acc_q.astype(jnp.bfloat16)
            lp2[q][...] = l_q
            r2[q] = [exchange(pack2[q], rctx2[q], link2[q], 4 + q),
                     exchange(lp2[q], rl2[q], link2[q], 12 + q)]
            for rdma in r2[q]:
                rdma.start()

        ldw.wait()
        wob = wo_ref[...].astype(jnp.bfloat16)

        for q in range(4):
            for rdma in r2[q]:
                rdma.wait()
            acc_q = (pack2[q][...].astype(jnp.float32)
                     + rctx2[q][...].astype(jnp.float32))
            l_q = l_ref[:, rows(q)] + rl2[q][...]
            ctx_q = (acc_q.reshape(G, HQ, DH)
                     / jnp.transpose(l_q)[:, :, None]).reshape(G, D)
            out_q = lax.dot_general(
                ctx_q.astype(jnp.bfloat16), wob, (((1,), (0,)), ((), ())),
                preferred_element_type=jnp.float32)
            out_vm[rows(q), :] = out_q
            for a in range(4):
                cp = pltpu.make_async_copy(
                    out_vm.at[pl.ds(q * G + 64 * a, 64)],
                    out_hbm.at[0, pl.ds(256 * a + 64 * q, 64)],
                    out_sems.at[4 * q + a])
                cp.start()
                outcp.append(cp)

        for cp in outcp:
            cp.wait()

    return pl.pallas_call(
        body,
        out_shape=jax.ShapeDtypeStruct((1, SQ, D), jnp.float32),
        in_specs=[
            pl.BlockSpec(memory_space=pltpu.MemorySpace.VMEM),
            pl.BlockSpec(memory_space=pltpu.MemorySpace.VMEM),
            pl.BlockSpec(memory_space=pltpu.MemorySpace.HBM),
            pl.BlockSpec(memory_space=pltpu.MemorySpace.HBM),
            pl.BlockSpec(memory_space=pltpu.MemorySpace.HBM),
        ],
        out_specs=pl.BlockSpec(memory_space=pltpu.MemorySpace.HBM),
        scratch_shapes=(
            [
                pltpu.VMEM((D, D), jnp.float32),
                pltpu.VMEM((HQ, SQ), jnp.float32),
                pltpu.VMEM((SQ, D), jnp.float32),
            ]
            + [pltpu.VMEM((G, HQ, DH), jnp.float32) for _ in range(8)]
            + [pltpu.VMEM((G, D), jnp.bfloat16) for _ in range(8)]
            + [pltpu.VMEM((HQ, G), jnp.float32) for _ in range(8)]
            + [pltpu.VMEM((G, D), jnp.bfloat16) for _ in range(8)]
            + [pltpu.VMEM((HQ, G), jnp.float32) for _ in range(8)]
            + [
                pltpu.SemaphoreType.DMA((3,)),
                pltpu.SemaphoreType.DMA((32,)),
                pltpu.SemaphoreType.DMA((16,)),
                pltpu.SemaphoreType.DMA((16,)),
                pltpu.SemaphoreType.DMA((16,)),
            ]
        ),
        compiler_params=pltpu.CompilerParams(
            collective_id=0, vmem_limit_bytes=64 * 1024 * 1024),
    )(x, Wq, K_ext, V_ext, Wo)


# device time: 44166 ns/iter; 1.3232x vs baseline; 1.0090x over previous
import jax
import jax.numpy as jnp
from jax import lax
from jax.experimental import pallas as pl
from jax.experimental.pallas import tpu as pltpu

N_DEV = 4
SQ = 1024
SKV = 1024
HQ = 8
DH = 128
D = HQ * DH
G = SQ // 4
SCALE = 0.08838834764831843


def kernel(x, Wq, K_ext, V_ext, Wo):
    def body(x_ref, wq_ref, k_hbm, v_hbm, wo_hbm, out_hbm, *scr):
        wo_ref, l_ref, out_vm = scr[0:3]
        kq = scr[3:7]
        vq = scr[7:11]
        pack1 = scr[11:15]
        pack2 = scr[15:19]
        lp1 = scr[19:23]
        lp2 = scr[23:27]
        rctx1 = scr[27:31]
        rctx2 = scr[31:35]
        rl1 = scr[35:39]
        rl2 = scr[39:43]
        load_sems, kv_sems, send_sems, recv_sems, out_sems = scr[43:48]

        p = lax.axis_index("i")
        p_y = p ^ 1
        p_x = 3 - p

        ldkv = [[], [], [], []]
        for q in range(4):
            for a in range(4):
                ck = pltpu.make_async_copy(
                    k_hbm.at[0, pl.ds(64 * (4 * a + q), 64)],
                    kq[q].at[pl.ds(64 * a, 64)],
                    kv_sems.at[8 * q + a])
                cv = pltpu.make_async_copy(
                    v_hbm.at[0, pl.ds(64 * (4 * a + q), 64)],
                    vq[q].at[pl.ds(64 * a, 64)],
                    kv_sems.at[8 * q + 4 + a])
                ck.start()
                cv.start()
                ldkv[q] += [ck, cv]
        ldw = pltpu.make_async_copy(wo_hbm, wo_ref, load_sems.at[2])
        ldw.start()

        bar = pltpu.get_barrier_semaphore()
        for nbr in (p_x, p_y):
            pl.semaphore_signal(bar, inc=1, device_id=(nbr,),
                                device_id_type=pl.DeviceIdType.MESH)
        pl.semaphore_wait(bar, 2)

        def exchange(src, dst, partner, sem_idx):
            return pltpu.make_async_remote_copy(
                src_ref=src, dst_ref=dst,
                send_sem=send_sems.at[sem_idx],
                recv_sem=recv_sems.at[sem_idx],
                device_id=(partner,),
                device_id_type=pl.DeviceIdType.MESH,
            )

        wqb = wq_ref[...].astype(jnp.bfloat16)

        link1 = [p_x, p_y, p_x, p_y]
        link2 = [p_y, p_x, p_y, p_x]
        r1 = [None] * 4
        r2 = [None] * 4
        outcp = []

        def rows(q):
            return pl.ds(q * G, G)

        for q in range(4):
            xq = jnp.concatenate(
                [x_ref[0, 64 * (4 * a + q):64 * (4 * a + q + 1), :]
                 for a in range(4)]).astype(jnp.bfloat16)
            qq = lax.dot_general(xq, wqb,
                                 (((1,), (0,)), ((), ())),
                                 preferred_element_type=jnp.float32)
            qqb = (qq * SCALE).astype(jnp.bfloat16)
            for cp in ldkv[q]:
                cp.wait()
            kbq = kq[q][...].reshape(G, D).astype(jnp.bfloat16)
            vbq = vq[q][...].reshape(G, D).astype(jnp.bfloat16)
            lrows = []
            for h in range(HQ):
                cols = pl.ds(h * DH, DH)
                kh = kbq[:, h * DH:(h + 1) * DH]
                vh = vbq[:, h * DH:(h + 1) * DH]
                s = lax.dot_general(qqb[:, h * DH:(h + 1) * DH], kh,
                                    (((1,), (1,)), ((), ())),
                                    preferred_element_type=jnp.float32)
                w = jnp.exp(s)
                lrows.append(w.sum(axis=1))
                ctx = lax.dot_general(
                    w.astype(jnp.bfloat16), vh, (((1,), (0,)), ((), ())),
                    preferred_element_type=jnp.float32)
                pack1[q][:, cols] = ctx.astype(jnp.bfloat16)
            l_q = jnp.stack(lrows)
            l_ref[:, rows(q)] = l_q
            lp1[q][...] = l_q
            r1[q] = [exchange(pack1[q], rctx1[q], link1[q], q),
                     exchange(lp1[q], rl1[q], link1[q], 8 + q)]
            for rdma in r1[q]:
                rdma.start()

        for q in range(4):
            for rdma in r1[q]:
                rdma.wait()
            acc_q = (pack1[q][...].astype(jnp.float32)
                     + rctx1[q][...].astype(jnp.float32))
            l_q = l_ref[:, rows(q)] + rl1[q][...]
            l_ref[:, rows(q)] = l_q
            pack2[q][...] = acc_q.astype(jnp.bfloat16)
            lp2[q][...] = l_q
            r2[q] = [exchange(pack2[q], rctx2[q], link2[q], 4 + q),
                     exchange(lp2[q], rl2[q], link2[q], 12 + q)]
            for rdma in r2[q]:
                rdma.start()

        ldw.wait()
        wob = wo_ref[...].astype(jnp.bfloat16)

        for q in range(4):
            for rdma in r2[q]:
                rdma.wait()
            acc_q = (pack2[q][...].astype(jnp.float32)
                     + rctx2[q][...].astype(jnp.float32))
            l_q = l_ref[:, rows(q)] + rl2[q][...]
            ctx_q = (acc_q.reshape(G, HQ, DH)
                     / jnp.transpose(l_q)[:, :, None]).reshape(G, D)
            out_q = lax.dot_general(
                ctx_q.astype(jnp.bfloat16), wob, (((1,), (0,)), ((), ())),
                preferred_element_type=jnp.float32)
            out_vm[rows(q), :] = out_q
            for a in range(4):
                cp = pltpu.make_async_copy(
                    out_vm.at[pl.ds(q * G + 64 * a, 64)],
                    out_hbm.at[0, pl.ds(256 * a + 64 * q, 64)],
                    out_sems.at[4 * q + a])
                cp.start()
                outcp.append(cp)

        for cp in outcp:
            cp.wait()

    return pl.pallas_call(
        body,
        out_shape=jax.ShapeDtypeStruct((1, SQ, D), jnp.float32),
        in_specs=[
            pl.BlockSpec(memory_space=pltpu.MemorySpace.VMEM),
            pl.BlockSpec(memory_space=pltpu.MemorySpace.VMEM),
            pl.BlockSpec(memory_space=pltpu.MemorySpace.HBM),
            pl.BlockSpec(memory_space=pltpu.MemorySpace.HBM),
            pl.BlockSpec(memory_space=pltpu.MemorySpace.HBM),
        ],
        out_specs=pl.BlockSpec(memory_space=pltpu.MemorySpace.HBM),
        scratch_shapes=(
            [
                pltpu.VMEM((D, D), jnp.float32),
                pltpu.VMEM((HQ, SQ), jnp.float32),
                pltpu.VMEM((SQ, D), jnp.float32),
            ]
            + [pltpu.VMEM((G, HQ, DH), jnp.float32) for _ in range(8)]
            + [pltpu.VMEM((G, D), jnp.bfloat16) for _ in range(8)]
            + [pltpu.VMEM((HQ, G), jnp.float32) for _ in range(8)]
            + [pltpu.VMEM((G, D), jnp.bfloat16) for _ in range(8)]
            + [pltpu.VMEM((HQ, G), jnp.float32) for _ in range(8)]
            + [
                pltpu.SemaphoreType.DMA((3,)),
                pltpu.SemaphoreType.DMA((32,)),
                pltpu.SemaphoreType.DMA((16,)),
                pltpu.SemaphoreType.DMA((16,)),
                pltpu.SemaphoreType.DMA((16,)),
            ]
        ),
        compiler_params=pltpu.CompilerParams(
            collective_id=0, vmem_limit_bytes=64 * 1024 * 1024),
    )(x, Wq, K_ext, V_ext, Wo)
